# baseline (speedup 1.0000x reference)
import jax
import jax.numpy as jnp
from jax.experimental import pallas as pl


_BN = 8192


def _body(x_ref, w_ref, b_ref, o_ref):
    o_ref[...] = jax.lax.dot_general(
        x_ref[...], w_ref[...],
        dimension_numbers=(((1,), (1,)), ((), ())),
        preferred_element_type=jnp.float32,
    ) + b_ref[...]


def kernel(x, tgt, table_w0, table_b0):
    del tgt
    B, I, H = x.shape
    N = table_w0.shape[0]
    x2 = x.reshape(B * I, H)
    b2 = table_b0.reshape(1, N)
    out = pl.pallas_call(
        _body,
        grid=(pl.cdiv(N, _BN),),
        in_specs=[
            pl.BlockSpec((B * I, H), lambda i: (0, 0)),
            pl.BlockSpec((_BN, H), lambda i: (i, 0)),
            pl.BlockSpec((1, _BN), lambda i: (0, i)),
        ],
        out_specs=pl.BlockSpec((B * I, _BN), lambda i: (0, i)),
        out_shape=jax.ShapeDtypeStruct((B * I, N), jnp.float32),
    )(x2, table_w0, b2)
    return out.reshape(B, I, N)


# speedup vs baseline: 1.0499x; 1.0499x over previous
import jax
import jax.numpy as jnp
from jax.experimental import pallas as pl


_BN = 4096


def _body(x_ref, w_ref, b_ref, o_ref):
    o_ref[...] = jax.lax.dot_general(
        x_ref[...], w_ref[...],
        dimension_numbers=(((1,), (1,)), ((), ())),
        preferred_element_type=jnp.float32,
    ) + b_ref[...]


def kernel(x, tgt, table_w0, table_b0):
    del tgt
    B, I, H = x.shape
    N = table_w0.shape[0]
    x2 = x.reshape(B * I, H)
    b2 = table_b0.reshape(1, N)
    out = pl.pallas_call(
        _body,
        grid=(pl.cdiv(N, _BN),),
        in_specs=[
            pl.BlockSpec((B * I, H), lambda i: (0, 0)),
            pl.BlockSpec((_BN, H), lambda i: (i, 0)),
            pl.BlockSpec((1, _BN), lambda i: (0, i)),
        ],
        out_specs=pl.BlockSpec((B * I, _BN), lambda i: (0, i)),
        out_shape=jax.ShapeDtypeStruct((B * I, N), jnp.float32),
    )(x2, table_w0, b2)
    return out.reshape(B, I, N)
